# Initial kernel scaffold; baseline (speedup 1.0000x reference)
import jax
import jax.numpy as jnp
from jax.experimental import pallas as pl


def kernel(x, edge_index, batch, Wm0, bm0, Wu0, bu0, Wm1, bm1, Wu1, bu1, Wm2, bm2, Wu2, bu2, Wp, bp):
    raise NotImplementedError("write your pallas kernel here")


# speedup vs baseline: 2.6047x; 2.6047x over previous
import functools

import jax
import jax.numpy as jnp
from jax import lax
from jax.experimental import pallas as pl
from jax.experimental.pallas import tpu as pltpu
from jax.experimental.pallas import tpu_sc as plsc

N = 10000
E = 160000
D = 256
H = 256
EMB = 128
G = 64

BN = 1000
NB = N // BN
HALF = 128

NC = 2
NS = 16
EC = 128
EPT = 10240
CH = EPT // EC
E_PAD = NS * EPT
RPT = 624
REM = N - NS * RPT


def _msg_body(h_ref, wm_ref, bm_ref, wua_ref, bu_ref, m_ref, u_ref):
    h = h_ref[...]
    m = jnp.dot(h, wm_ref[...], preferred_element_type=jnp.float32) + bm_ref[0]
    m_ref[0] = m[:, :HALF]
    m_ref[1] = m[:, HALF:]
    u_ref[...] = jnp.dot(h, wua_ref[...], preferred_element_type=jnp.float32) + bu_ref[0]


_msg_call = pl.pallas_call(
    _msg_body,
    grid=(NB,),
    in_specs=[
        pl.BlockSpec((BN, H), lambda i: (i, 0)),
        pl.BlockSpec((H, H), lambda i: (0, 0)),
        pl.BlockSpec((1, H), lambda i: (0, 0)),
        pl.BlockSpec((H, H), lambda i: (0, 0)),
        pl.BlockSpec((1, H), lambda i: (0, 0)),
    ],
    out_specs=[
        pl.BlockSpec((2, BN, HALF), lambda i: (0, i, 0)),
        pl.BlockSpec((BN, H), lambda i: (i, 0)),
    ],
    out_shape=[
        jax.ShapeDtypeStruct((2, N, HALF), jnp.float32),
        jax.ShapeDtypeStruct((N, H), jnp.float32),
    ],
)


def _upd_body(u_ref, s_ref, wub_ref, o_ref):
    acc = u_ref[...]
    acc += jnp.dot(s_ref[0], wub_ref[0], preferred_element_type=jnp.float32)
    acc += jnp.dot(s_ref[1], wub_ref[1], preferred_element_type=jnp.float32)
    o_ref[...] = jnp.maximum(acc, 0.0)


_upd_call = pl.pallas_call(
    _upd_body,
    grid=(NB,),
    in_specs=[
        pl.BlockSpec((BN, H), lambda i: (i, 0)),
        pl.BlockSpec((2, BN, HALF), lambda i: (0, i, 0)),
        pl.BlockSpec((2, HALF, H), lambda i: (0, 0, 0)),
    ],
    out_specs=pl.BlockSpec((BN, H), lambda i: (i, 0)),
    out_shape=jax.ShapeDtypeStruct((N, H), jnp.float32),
)


def _pool_body(h_ref, wp_ref, bp_ref, b_ref, o_ref, sums, cnts):
    i = pl.program_id(0)

    @pl.when(i == 0)
    def _():
        sums[...] = jnp.zeros_like(sums)
        cnts[...] = jnp.zeros_like(cnts)

    z = jnp.tanh(
        jnp.dot(h_ref[...], wp_ref[...], preferred_element_type=jnp.float32)
        + bp_ref[0]
    )
    gids = lax.broadcasted_iota(jnp.int32, (BN, G), 1)
    onehot = jnp.where(b_ref[...] == gids, 1.0, 0.0)
    dnums = (((0,), (0,)), ((), ()))
    sums[...] += lax.dot_general(onehot, z, dnums, preferred_element_type=jnp.float32)
    cnts[...] += lax.dot_general(
        onehot, jnp.ones((BN, EMB), jnp.float32), dnums,
        preferred_element_type=jnp.float32)

    @pl.when(i == NB - 1)
    def _():
        o_ref[...] = sums[...] / jnp.maximum(cnts[...], 1.0)


_pool_call = pl.pallas_call(
    _pool_body,
    grid=(NB,),
    in_specs=[
        pl.BlockSpec((BN, H), lambda i: (i, 0)),
        pl.BlockSpec((H, EMB), lambda i: (0, 0)),
        pl.BlockSpec((1, EMB), lambda i: (0, 0)),
        pl.BlockSpec((BN, 1), lambda i: (i, 0)),
    ],
    out_specs=pl.BlockSpec((G, EMB), lambda i: (0, 0)),
    out_shape=jax.ShapeDtypeStruct((G, EMB), jnp.float32),
    scratch_shapes=[
        pltpu.VMEM((G, EMB), jnp.float32),
        pltpu.VMEM((G, EMB), jnp.float32),
    ],
)


def _segsum_body(m_hbm, src2_hbm, dst_hbm, out_hbm, sidx, didx, rows, acc, sem):
    c = lax.axis_index("c")
    s = lax.axis_index("s")

    def _zrow(r, carry):
        for k in range(8):
            rows[r, pl.ds(k * 16, 16)] = jnp.zeros((16,), jnp.float32)
        return carry

    lax.fori_loop(0, EC, _zrow, 0)
    base_row = s * RPT
    for t in range(RPT // EC):
        pltpu.sync_copy(rows, acc.at[pl.ds(base_row + t * EC, EC)])
    rem = RPT % EC
    pltpu.sync_copy(rows.at[pl.ds(0, rem)],
                    acc.at[pl.ds(base_row + (RPT // EC) * EC, rem)])

    @pl.when(s == NS - 1)
    def _():
        pltpu.sync_copy(rows.at[pl.ds(0, REM)],
                        acc.at[pl.ds(NS * RPT, REM)])

    plsc.subcore_barrier()

    ebase = s * EPT
    gbase = c * E_PAD + ebase

    def _chunk(j, carry):
        off = j * EC
        pltpu.sync_copy(src2_hbm.at[pl.ds(gbase + off, EC)], sidx)
        pltpu.sync_copy(dst_hbm.at[pl.ds(ebase + off, EC)], didx)
        pltpu.async_copy(m_hbm.at[sidx], rows, sem).wait()
        pltpu.sync_copy(rows, acc.at[didx], add=True)
        return carry

    lax.fori_loop(0, CH, _chunk, 0)
    plsc.subcore_barrier()

    pltpu.sync_copy(acc.at[pl.ds(base_row, RPT)],
                    out_hbm.at[pl.ds(c * N + base_row, RPT)])

    @pl.when(s == NS - 1)
    def _():
        pltpu.sync_copy(acc.at[pl.ds(NS * RPT, REM)],
                        out_hbm.at[pl.ds(c * N + NS * RPT, REM)])


@functools.lru_cache(maxsize=None)
def _segsum_call():
    return functools.partial(
        pl.kernel,
        out_type=jax.ShapeDtypeStruct((2 * N, HALF), jnp.float32),
        mesh=plsc.VectorSubcoreMesh(
            core_axis_name="c", subcore_axis_name="s", num_cores=NC,
            num_subcores=NS),
        scratch_types=[
            pltpu.VMEM((EC,), jnp.int32),
            pltpu.VMEM((EC,), jnp.int32),
            pltpu.VMEM((EC, HALF), jnp.float32),
            pltpu.VMEM_SHARED((N + 8, HALF), jnp.float32),
            pltpu.SemaphoreType.DMA,
        ],
    )(_segsum_body)


def kernel(x, edge_index, batch,
           Wm0, bm0, Wu0, bu0,
           Wm1, bm1, Wu1, bu1,
           Wm2, bm2, Wu2, bu2,
           Wp, bp):
    src = edge_index[0]
    dst = edge_index[1]
    pad = E_PAD - E
    src_p = jnp.concatenate([src, jnp.zeros((pad,), jnp.int32)])
    dst_p = jnp.concatenate([dst, jnp.full((pad,), N, jnp.int32)])
    src2 = jnp.concatenate([src_p, src_p + N])

    h = x
    for Wm, bm, Wu, bu in ((Wm0, bm0, Wu0, bu0),
                           (Wm1, bm1, Wu1, bu1),
                           (Wm2, bm2, Wu2, bu2)):
        m2, u = _msg_call(h, Wm, bm.reshape(1, H),
                          Wu[:H], bu.reshape(1, H))
        s = _segsum_call()(m2.reshape(2 * N, HALF), src2, dst_p)
        h = _upd_call(u, s.reshape(2, N, HALF),
                      Wu[H:].reshape(2, HALF, H))

    return _pool_call(h, Wp, bp.reshape(1, EMB), batch.reshape(N, 1))
